# baseline (speedup 1.0000x reference)
import functools

import jax
import jax.numpy as jnp
from jax import lax
from jax.experimental import pallas as pl
from jax.experimental.pallas import tpu as pltpu
from jax.experimental.pallas import tpu_sc as plsc

_P = 16
_HB = 4
_MASK_RATIO = 0.6


def _stats_kernel(inp_ref, rec_ref, out_ref):
    S = inp_ref.shape[2]
    P = _P
    Fp = S // P

    r_iota = lax.broadcasted_iota(jnp.int32, (Fp, S), 1)
    g_iota = lax.broadcasted_iota(jnp.int32, (Fp, S), 0)
    MwT = (r_iota // P == g_iota).astype(jnp.float32)
    Md = jnp.transpose(MwT)

    def pool(v):
        t = jnp.dot(MwT, v, preferred_element_type=jnp.float32)
        return jnp.dot(t, Md, preferred_element_type=jnp.float32)

    n = float(P * P * P)
    for h in range(_HB):
        s1 = s2 = r1 = r2 = c = None
        for k in range(P):
            xk = inp_ref[0, h * P + k]
            yk = rec_ref[0, h * P + k]
            if k == 0:
                s1, s2, r1, r2, c = xk, xk * xk, yk, yk * yk, xk * yk
            else:
                s1 = s1 + xk
                s2 = s2 + xk * xk
                r1 = r1 + yk
                r2 = r2 + yk * yk
                c = c + xk * yk
        s1, s2, r1, r2, c = pool(s1), pool(s2), pool(r1), pool(r2), pool(c)

        mean = s1 / n
        var = s2 / n - mean * mean
        std = jnp.sqrt(var + 1e-6)
        l2 = r2 / n - 2.0 * (c / n - mean * (r1 / n)) / std + var / (var + 1e-6)
        out_ref[0, h] = l2


def _make_sc_mask(B, L, len_keep):
    NV = L // 16
    mesh = plsc.VectorSubcoreMesh(core_axis_name="c", subcore_axis_name="s")
    info = plsc.get_sparse_core_info()
    nc = info.num_cores

    @functools.partial(
        pl.kernel,
        out_type=jax.ShapeDtypeStruct((B, L), jnp.float32),
        scratch_types=[
            pltpu.VMEM((L,), jnp.int32),
            pltpu.VMEM((L,), jnp.float32),
        ],
        mesh=mesh,
    )
    def mask_kernel(bits_hbm, out_hbm, row_v, na_v):
        wid = lax.axis_index("s") * nc + lax.axis_index("c")

        @pl.when(wid < B)
        def _():
            pltpu.sync_copy(bits_hbm.at[wid], row_v)
            lane = lax.iota(jnp.int32, 16)

            def lane_total(x):
                acc = x
                for d in (1, 2, 4, 8):
                    idx = jnp.bitwise_and(lane + d, 15)
                    acc = acc + acc.at[idx].get(mode="promise_in_bounds")
                return acc

            vs = [row_v[pl.ds(u * 16, 16)] for u in range(NV)]

            def cnt_le(mid):
                part = jnp.zeros((16,), jnp.int32)
                for u in range(NV):
                    part = part + jnp.where(vs[u] <= mid, 1, 0).astype(
                        jnp.int32
                    )
                return lane_total(part)

            def bs(_, carry):
                lo, hi = carry
                mid = lo + lax.shift_right_arithmetic(hi - lo, 1)
                take_low = cnt_le(mid) >= len_keep
                return (
                    jnp.where(take_low, lo, mid + 1),
                    jnp.where(take_low, mid, hi),
                )

            t, _unused = lax.fori_loop(
                0,
                31,
                bs,
                (
                    jnp.zeros((16,), jnp.int32),
                    jnp.full((16,), 2**31 - 1, jnp.int32),
                ),
            )

            part_lt = jnp.zeros((16,), jnp.int32)
            for u in range(NV):
                part_lt = part_lt + jnp.where(vs[u] < t, 1, 0).astype(
                    jnp.int32
                )
            m = lane_total(part_lt)
            cle = cnt_le(t)
            need = len_keep - m

            def fast_path():
                for u in range(NV):
                    na_v[pl.ds(u * 16, 16)] = jnp.where(vs[u] <= t, 0.0, 1.0)

            def slow_path():
                def body(u, prefix):
                    v = row_v[pl.ds(u * 16, 16)]
                    eq = v == t
                    eqi = jnp.where(eq, 1, 0).astype(jnp.int32)
                    ps = eqi
                    for d in (1, 2, 4, 8):
                        idx = jnp.maximum(lane - d, 0)
                        shifted = ps.at[idx].get(mode="promise_in_bounds")
                        ps = ps + jnp.where(lane >= d, shifted, 0)
                    rank_eq = prefix + (ps - eqi)
                    active = (v < t) | (eq & (rank_eq < need))
                    na_v[pl.ds(u * 16, 16)] = jnp.where(active, 0.0, 1.0)
                    return prefix + lane_total(eqi)

                lax.fori_loop(0, NV, body, jnp.zeros((16,), jnp.int32))

            lax.cond(cle[0] == len_keep, fast_path, slow_path)
            pltpu.sync_copy(na_v, out_hbm.at[wid])

    return mask_kernel


def _loss_kernel(l2_ref, na_ref, out_ref):
    masked = l2_ref[:, :] * na_ref[:, :]
    num = jnp.sum(masked, keepdims=True).reshape(1, 1)
    den = jnp.sum(na_ref[:, :], keepdims=True).reshape(1, 1)
    out_ref[:, :] = num / (den + 1e-8)


def kernel(inp_bchwd, rec_bchwd, scores):
    B, C, S = inp_bchwd.shape[0], inp_bchwd.shape[1], inp_bchwd.shape[2]
    P = _P
    Fp = S // P
    L = Fp * Fp * Fp
    len_keep = int(round(L * (1.0 - _MASK_RATIO)))

    inp3 = inp_bchwd.reshape(B, S, S, S)
    rec3 = rec_bchwd.reshape(B, S, S, S)

    bits = lax.bitcast_convert_type(scores, jnp.int32)
    nonact = _make_sc_mask(B, L, len_keep)(bits)

    l2 = pl.pallas_call(
        _stats_kernel,
        grid=(B, Fp // _HB),
        in_specs=[
            pl.BlockSpec((1, _HB * P, S, S), lambda b, h: (b, h, 0, 0)),
            pl.BlockSpec((1, _HB * P, S, S), lambda b, h: (b, h, 0, 0)),
        ],
        out_specs=pl.BlockSpec((1, _HB, Fp, Fp), lambda b, h: (b, h, 0, 0)),
        out_shape=jax.ShapeDtypeStruct((B, Fp, Fp, Fp), jnp.float32),
        compiler_params=pltpu.CompilerParams(
            dimension_semantics=("parallel", "parallel")),
    )(inp3, rec3)

    l2 = l2.reshape(B, L)

    loss = pl.pallas_call(
        _loss_kernel,
        out_shape=jax.ShapeDtypeStruct((1, 1), jnp.float32),
    )(l2, nonact)

    return loss[0, 0]


# speedup vs baseline: 1.0313x; 1.0313x over previous
import functools

import jax
import jax.numpy as jnp
from jax import lax
from jax.experimental import pallas as pl
from jax.experimental.pallas import tpu as pltpu
from jax.experimental.pallas import tpu_sc as plsc

_P = 16
_HB = 4
_MASK_RATIO = 0.6


def _stats_kernel(inp_ref, rec_ref, out_ref):
    S = inp_ref.shape[2]
    P = _P
    Fp = S // P

    r_iota = lax.broadcasted_iota(jnp.int32, (Fp, S), 1)
    g_iota = lax.broadcasted_iota(jnp.int32, (Fp, S), 0)
    MwT = (r_iota // P == g_iota).astype(jnp.float32)
    Md = jnp.transpose(MwT)

    def pool(v):
        t = jnp.dot(MwT, v, preferred_element_type=jnp.float32)
        return jnp.dot(t, Md, preferred_element_type=jnp.float32)

    n = float(P * P * P)
    for h in range(_HB):
        s1 = s2 = r1 = r2 = c = None
        for k in range(P):
            xk = inp_ref[0, h * P + k]
            yk = rec_ref[0, h * P + k]
            if k == 0:
                s1, s2, r1, r2, c = xk, xk * xk, yk, yk * yk, xk * yk
            else:
                s1 = s1 + xk
                s2 = s2 + xk * xk
                r1 = r1 + yk
                r2 = r2 + yk * yk
                c = c + xk * yk
        s1, s2, r1, r2, c = pool(s1), pool(s2), pool(r1), pool(r2), pool(c)

        mean = s1 / n
        var = s2 / n - mean * mean
        std = jnp.sqrt(var + 1e-6)
        l2 = r2 / n - 2.0 * (c / n - mean * (r1 / n)) / std + var / (var + 1e-6)
        out_ref[0, h] = l2


def _make_sc_mask(B, L, len_keep):
    NV = L // 16
    mesh = plsc.VectorSubcoreMesh(core_axis_name="c", subcore_axis_name="s", num_cores=1)
    info = plsc.get_sparse_core_info()
    nc = info.num_cores

    @functools.partial(
        pl.kernel,
        out_type=jax.ShapeDtypeStruct((B, L), jnp.float32),
        scratch_types=[
            pltpu.VMEM((L,), jnp.int32),
            pltpu.VMEM((L,), jnp.float32),
        ],
        mesh=mesh,
    )
    def mask_kernel(bits_hbm, out_hbm, row_v, na_v):
        wid = lax.axis_index("s") * nc + lax.axis_index("c")

        @pl.when(wid < B)
        def _():
            pltpu.sync_copy(bits_hbm.at[wid], row_v)
            lane = lax.iota(jnp.int32, 16)

            def lane_total(x):
                acc = x
                for d in (1, 2, 4, 8):
                    idx = jnp.bitwise_and(lane + d, 15)
                    acc = acc + acc.at[idx].get(mode="promise_in_bounds")
                return acc

            vs = [row_v[pl.ds(u * 16, 16)] for u in range(NV)]

            def cnt_le(mid):
                part = jnp.zeros((16,), jnp.int32)
                for u in range(NV):
                    part = part + jnp.where(vs[u] <= mid, 1, 0).astype(
                        jnp.int32
                    )
                return lane_total(part)

            def bs(_, carry):
                lo, hi = carry
                mid = lo + lax.shift_right_arithmetic(hi - lo, 1)
                take_low = cnt_le(mid) >= len_keep
                return (
                    jnp.where(take_low, lo, mid + 1),
                    jnp.where(take_low, mid, hi),
                )

            t, _unused = lax.fori_loop(
                0,
                31,
                bs,
                (
                    jnp.zeros((16,), jnp.int32),
                    jnp.full((16,), 2**31 - 1, jnp.int32),
                ),
            )

            part_lt = jnp.zeros((16,), jnp.int32)
            for u in range(NV):
                part_lt = part_lt + jnp.where(vs[u] < t, 1, 0).astype(
                    jnp.int32
                )
            m = lane_total(part_lt)
            cle = cnt_le(t)
            need = len_keep - m

            def fast_path():
                for u in range(NV):
                    na_v[pl.ds(u * 16, 16)] = jnp.where(vs[u] <= t, 0.0, 1.0)

            def slow_path():
                def body(u, prefix):
                    v = row_v[pl.ds(u * 16, 16)]
                    eq = v == t
                    eqi = jnp.where(eq, 1, 0).astype(jnp.int32)
                    ps = eqi
                    for d in (1, 2, 4, 8):
                        idx = jnp.maximum(lane - d, 0)
                        shifted = ps.at[idx].get(mode="promise_in_bounds")
                        ps = ps + jnp.where(lane >= d, shifted, 0)
                    rank_eq = prefix + (ps - eqi)
                    active = (v < t) | (eq & (rank_eq < need))
                    na_v[pl.ds(u * 16, 16)] = jnp.where(active, 0.0, 1.0)
                    return prefix + lane_total(eqi)

                lax.fori_loop(0, NV, body, jnp.zeros((16,), jnp.int32))

            lax.cond(cle[0] == len_keep, fast_path, slow_path)
            pltpu.sync_copy(na_v, out_hbm.at[wid])

    return mask_kernel


def _loss_kernel(l2_ref, na_ref, out_ref):
    masked = l2_ref[:, :] * na_ref[:, :]
    num = jnp.sum(masked, keepdims=True).reshape(1, 1)
    den = jnp.sum(na_ref[:, :], keepdims=True).reshape(1, 1)
    out_ref[:, :] = num / (den + 1e-8)


def kernel(inp_bchwd, rec_bchwd, scores):
    B, C, S = inp_bchwd.shape[0], inp_bchwd.shape[1], inp_bchwd.shape[2]
    P = _P
    Fp = S // P
    L = Fp * Fp * Fp
    len_keep = int(round(L * (1.0 - _MASK_RATIO)))

    inp3 = inp_bchwd.reshape(B, S, S, S)
    rec3 = rec_bchwd.reshape(B, S, S, S)

    bits = lax.bitcast_convert_type(scores, jnp.int32)
    nonact = _make_sc_mask(B, L, len_keep)(bits)

    l2 = pl.pallas_call(
        _stats_kernel,
        grid=(B, Fp // _HB),
        in_specs=[
            pl.BlockSpec((1, _HB * P, S, S), lambda b, h: (b, h, 0, 0)),
            pl.BlockSpec((1, _HB * P, S, S), lambda b, h: (b, h, 0, 0)),
        ],
        out_specs=pl.BlockSpec((1, _HB, Fp, Fp), lambda b, h: (b, h, 0, 0)),
        out_shape=jax.ShapeDtypeStruct((B, Fp, Fp, Fp), jnp.float32),
        compiler_params=pltpu.CompilerParams(
            dimension_semantics=("parallel", "parallel")),
    )(inp3, rec3)

    l2 = l2.reshape(B, L)

    loss = pl.pallas_call(
        _loss_kernel,
        out_shape=jax.ShapeDtypeStruct((1, 1), jnp.float32),
    )(l2, nonact)

    return loss[0, 0]


# speedup vs baseline: 1.0324x; 1.0011x over previous
import functools

import jax
import jax.numpy as jnp
from jax import lax
from jax.experimental import pallas as pl
from jax.experimental.pallas import tpu as pltpu
from jax.experimental.pallas import tpu_sc as plsc

_P = 16
_HB = 4
_MASK_RATIO = 0.6


def _stats_kernel(inp_ref, rec_ref, out_ref):
    S = inp_ref.shape[2]
    P = _P
    Fp = S // P

    r_iota = lax.broadcasted_iota(jnp.int32, (Fp, S), 1)
    g_iota = lax.broadcasted_iota(jnp.int32, (Fp, S), 0)
    MwT = (r_iota // P == g_iota).astype(jnp.float32)
    Md = jnp.transpose(MwT)

    def pool(v):
        t = jnp.dot(MwT, v, preferred_element_type=jnp.float32)
        return jnp.dot(t, Md, preferred_element_type=jnp.float32)

    n = float(P * P * P)
    for h in range(_HB):
        s1 = s2 = r1 = r2 = c = None
        for k in range(P):
            xk = inp_ref[0, h * P + k]
            yk = rec_ref[0, h * P + k]
            if k == 0:
                s1, s2, r1, r2, c = xk, xk * xk, yk, yk * yk, xk * yk
            else:
                s1 = s1 + xk
                s2 = s2 + xk * xk
                r1 = r1 + yk
                r2 = r2 + yk * yk
                c = c + xk * yk
        s1, s2, r1, r2, c = pool(s1), pool(s2), pool(r1), pool(r2), pool(c)

        mean = s1 / n
        var = s2 / n - mean * mean
        std = jnp.sqrt(var + 1e-6)
        l2 = r2 / n - 2.0 * (c / n - mean * (r1 / n)) / std + var / (var + 1e-6)
        out_ref[0, h] = l2


def _make_sc_mask(B, L, len_keep):
    NV = L // 16
    nc = 1
    mesh = plsc.VectorSubcoreMesh(
        core_axis_name="c", subcore_axis_name="s", num_cores=nc
    )

    @functools.partial(
        pl.kernel,
        out_type=jax.ShapeDtypeStruct((B, L), jnp.float32),
        scratch_types=[
            pltpu.VMEM((L,), jnp.int32),
            pltpu.VMEM((L,), jnp.float32),
        ],
        mesh=mesh,
    )
    def mask_kernel(bits_hbm, out_hbm, row_v, na_v):
        wid = lax.axis_index("s") * nc + lax.axis_index("c")

        @pl.when(wid < B)
        def _():
            pltpu.sync_copy(bits_hbm.at[wid], row_v)
            lane = lax.iota(jnp.int32, 16)

            def lane_total(x):
                acc = x
                for d in (1, 2, 4, 8):
                    idx = jnp.bitwise_and(lane + d, 15)
                    acc = acc + acc.at[idx].get(mode="promise_in_bounds")
                return acc

            vs = [row_v[pl.ds(u * 16, 16)] for u in range(NV)]

            def cnt_le(mid):
                part = jnp.zeros((16,), jnp.int32)
                for u in range(NV):
                    part = part + jnp.where(vs[u] <= mid, 1, 0).astype(
                        jnp.int32
                    )
                return lane_total(part)

            def bs(_, carry):
                lo, hi = carry
                mid = lo + lax.shift_right_arithmetic(hi - lo, 1)
                take_low = cnt_le(mid) >= len_keep
                return (
                    jnp.where(take_low, lo, mid + 1),
                    jnp.where(take_low, mid, hi),
                )

            t, _unused = lax.fori_loop(
                0,
                31,
                bs,
                (
                    jnp.zeros((16,), jnp.int32),
                    jnp.full((16,), 2**31 - 1, jnp.int32),
                ),
            )

            part_lt = jnp.zeros((16,), jnp.int32)
            for u in range(NV):
                part_lt = part_lt + jnp.where(vs[u] < t, 1, 0).astype(
                    jnp.int32
                )
            m = lane_total(part_lt)
            cle = cnt_le(t)
            need = len_keep - m

            def fast_path():
                for u in range(NV):
                    na_v[pl.ds(u * 16, 16)] = jnp.where(vs[u] <= t, 0.0, 1.0)

            def slow_path():
                def body(u, prefix):
                    v = row_v[pl.ds(u * 16, 16)]
                    eq = v == t
                    eqi = jnp.where(eq, 1, 0).astype(jnp.int32)
                    ps = eqi
                    for d in (1, 2, 4, 8):
                        idx = jnp.maximum(lane - d, 0)
                        shifted = ps.at[idx].get(mode="promise_in_bounds")
                        ps = ps + jnp.where(lane >= d, shifted, 0)
                    rank_eq = prefix + (ps - eqi)
                    active = (v < t) | (eq & (rank_eq < need))
                    na_v[pl.ds(u * 16, 16)] = jnp.where(active, 0.0, 1.0)
                    return prefix + lane_total(eqi)

                lax.fori_loop(0, NV, body, jnp.zeros((16,), jnp.int32))

            lax.cond(cle[0] == len_keep, fast_path, slow_path)
            pltpu.sync_copy(na_v, out_hbm.at[wid])

    return mask_kernel


def _loss_kernel(l2_ref, na_ref, out_ref):
    masked = l2_ref[:, :] * na_ref[:, :]
    num = jnp.sum(masked, keepdims=True).reshape(1, 1)
    den = jnp.sum(na_ref[:, :], keepdims=True).reshape(1, 1)
    out_ref[:, :] = num / (den + 1e-8)


def kernel(inp_bchwd, rec_bchwd, scores):
    B, C, S = inp_bchwd.shape[0], inp_bchwd.shape[1], inp_bchwd.shape[2]
    P = _P
    Fp = S // P
    L = Fp * Fp * Fp
    len_keep = int(round(L * (1.0 - _MASK_RATIO)))

    inp3 = inp_bchwd.reshape(B, S, S, S)
    rec3 = rec_bchwd.reshape(B, S, S, S)

    bits = lax.bitcast_convert_type(scores, jnp.int32)
    nonact = _make_sc_mask(B, L, len_keep)(bits)

    l2 = pl.pallas_call(
        _stats_kernel,
        grid=(B, Fp // _HB),
        in_specs=[
            pl.BlockSpec((1, _HB * P, S, S), lambda b, h: (b, h, 0, 0)),
            pl.BlockSpec((1, _HB * P, S, S), lambda b, h: (b, h, 0, 0)),
        ],
        out_specs=pl.BlockSpec((1, _HB, Fp, Fp), lambda b, h: (b, h, 0, 0)),
        out_shape=jax.ShapeDtypeStruct((B, Fp, Fp, Fp), jnp.float32),
        compiler_params=pltpu.CompilerParams(
            dimension_semantics=("parallel", "parallel")),
    )(inp3, rec3)

    l2 = l2.reshape(B, L)

    loss = pl.pallas_call(
        _loss_kernel,
        out_shape=jax.ShapeDtypeStruct((1, 1), jnp.float32),
    )(l2, nonact)

    return loss[0, 0]
